# speedup vs baseline: 3.3210x; 3.3210x over previous
import jax
import jax.numpy as jnp
from jax import lax
from jax.experimental import pallas as pl
from jax.experimental.pallas import tpu as pltpu
from jax.experimental.pallas import tpu_sc as plsc

N = 10000
M = 160000
D = 256

NC = 2
NS = 16
NW = NC * NS
EPW = M // NW
CHUNK = 200
NCHUNK = EPW // CHUNK


def _proj_body(h_ref, wc_ref, o_ref):
    o_ref[...] = lax.dot_general(
        h_ref[...], wc_ref[...],
        dimension_numbers=(((1,), (1,)), ((), ())),
        preferred_element_type=jnp.float32,
    )


def _node_table(H, Wc):
    bn = 2000
    return pl.pallas_call(
        _proj_body,
        grid=(N // bn,),
        in_specs=[
            pl.BlockSpec((bn, D), lambda i: (i, 0)),
            pl.BlockSpec((2 * D, D), lambda i: (0, 0)),
        ],
        out_specs=pl.BlockSpec((bn, 2 * D), lambda i: (i, 0)),
        out_shape=jax.ShapeDtypeStruct((N, 2 * D), jnp.float32),
    )(H, Wc)


def _sc_gather_body(g_hbm, i0_hbm, i1_hbm, s_hbm,
                    i0_v, i1_v, rows0, rows1, sem0, sem1):
    wid = lax.axis_index("s") * NC + lax.axis_index("c")
    base = wid * EPW
    pltpu.sync_copy(i0_hbm.at[pl.ds(base, EPW)], i0_v)
    pltpu.sync_copy(i1_hbm.at[pl.ds(base, EPW)], i1_v)

    def chunk(j, carry):
        off = j * CHUNK
        cp0 = pltpu.async_copy(g_hbm.at[i0_v.at[pl.ds(off, CHUNK)]], rows0, sem0)
        cp1 = pltpu.async_copy(g_hbm.at[i1_v.at[pl.ds(off, CHUNK)]], rows1, sem1)
        cp0.wait()
        cp1.wait()

        def addrow(i, c):
            for k in range(D // 16):
                sl = pl.ds(k * 16, 16)
                rows0[i, sl] = rows0[i, sl] + rows1[i, sl]
            return c

        lax.fori_loop(0, CHUNK, addrow, 0, unroll=False)
        pltpu.sync_copy(rows0, s_hbm.at[pl.ds(base + off, CHUNK)])
        return carry

    lax.fori_loop(0, NCHUNK, chunk, 0, unroll=False)


def _sc_gather(G2, idx0, idx1):
    mesh = plsc.VectorSubcoreMesh(
        core_axis_name="c", subcore_axis_name="s",
        num_cores=NC, num_subcores=NS,
    )
    fn = pl.kernel(
        _sc_gather_body,
        out_type=jax.ShapeDtypeStruct((M, D), jnp.float32),
        mesh=mesh,
        scratch_types=[
            pltpu.VMEM((EPW,), jnp.int32),
            pltpu.VMEM((EPW,), jnp.int32),
            pltpu.VMEM((CHUNK, D), jnp.float32),
            pltpu.VMEM((CHUNK, D), jnp.float32),
            pltpu.SemaphoreType.DMA,
            pltpu.SemaphoreType.DMA,
        ],
    )
    return fn(G2, idx0, idx1)


def _edge_body(e_ref, s_ref, w2_ref, b_ref, lnw_ref, lnb_ref, o_ref):
    e = e_ref[...]
    f = lax.dot_general(
        e, w2_ref[...],
        dimension_numbers=(((1,), (1,)), ((), ())),
        preferred_element_type=jnp.float32,
    )
    pre = f + s_ref[...] + b_ref[...]
    act = jnp.where(pre >= 0, pre, 0.01 * pre)
    x = act + e
    mu = jnp.mean(x, axis=1, keepdims=True)
    xc = x - mu
    var = jnp.mean(xc * xc, axis=1, keepdims=True)
    inv = lax.rsqrt(var + 1e-5)
    o_ref[...] = xc * inv * lnw_ref[...] + lnb_ref[...]


def _edge_update(E, S, W2, b, ln_w, ln_b):
    bm = 1000
    return pl.pallas_call(
        _edge_body,
        grid=(M // bm,),
        in_specs=[
            pl.BlockSpec((bm, D), lambda i: (i, 0)),
            pl.BlockSpec((bm, D), lambda i: (i, 0)),
            pl.BlockSpec((D, D), lambda i: (0, 0)),
            pl.BlockSpec((1, D), lambda i: (0, 0)),
            pl.BlockSpec((1, D), lambda i: (0, 0)),
            pl.BlockSpec((1, D), lambda i: (0, 0)),
        ],
        out_specs=pl.BlockSpec((bm, D), lambda i: (i, 0)),
        out_shape=jax.ShapeDtypeStruct((M, D), jnp.float32),
    )(E, S, W2, b, ln_w, ln_b)


@jax.jit
def kernel(H, E, ht, W, b, ln_w, ln_b):
    W1 = W[:, :D]
    W2 = W[:, D:2 * D]
    W3 = W[:, 2 * D:]
    Wc = jnp.concatenate([W1, W3], axis=0)

    G = _node_table(H, Wc)
    G2 = G.reshape(2 * N, D)

    idx0 = ht[:, 0] * 2
    idx1 = ht[:, 1] * 2 + 1
    S = _sc_gather(G2, idx0, idx1)

    b2 = b.reshape(1, D)
    lnw2 = ln_w.reshape(1, D)
    lnb2 = ln_b.reshape(1, D)
    return _edge_update(E, S, W2, b2, lnw2, lnb2)


# speedup vs baseline: 4.5469x; 1.3692x over previous
import jax
import jax.numpy as jnp
from jax import lax
from jax.experimental import pallas as pl
from jax.experimental.pallas import tpu as pltpu
from jax.experimental.pallas import tpu_sc as plsc

N = 10000
M = 160000
D = 256

NC = 2
NS = 16
NW = NC * NS
EPW = M // NW
CHUNK = 200
NCHUNK = EPW // CHUNK


def _proj_body(h_ref, wc_ref, o_ref):
    p = lax.dot_general(
        h_ref[...], wc_ref[...],
        dimension_numbers=(((1,), (1,)), ((), ())),
        preferred_element_type=jnp.float32,
    )
    r = p.astype(jnp.bfloat16).astype(jnp.float32)
    u = lax.bitcast_convert_type(r, jnp.uint32)
    s16 = jnp.uint32(16)
    mask = jnp.uint32(0xFFFF0000)
    head = lax.shift_right_logical(u[:, 0:128], s16) | (u[:, 128:256] & mask)
    tail = lax.shift_right_logical(u[:, 256:384], s16) | (u[:, 384:512] & mask)
    o_ref[...] = lax.bitcast_convert_type(
        jnp.concatenate([head, tail], axis=1), jnp.int32)


def _node_table(H, Wc):
    bn = 2000
    return pl.pallas_call(
        _proj_body,
        grid=(N // bn,),
        in_specs=[
            pl.BlockSpec((bn, D), lambda i: (i, 0)),
            pl.BlockSpec((2 * D, D), lambda i: (0, 0)),
        ],
        out_specs=pl.BlockSpec((bn, D), lambda i: (i, 0)),
        out_shape=jax.ShapeDtypeStruct((N, D), jnp.int32),
    )(H, Wc)


def _sc_gather_body(g_hbm, i0_hbm, i1_hbm, s0_hbm, s1_hbm,
                    i0_v, i1_v, rows0, rows1, sem0, sem1):
    wid = lax.axis_index("s") * NC + lax.axis_index("c")
    base = wid * EPW
    pltpu.sync_copy(i0_hbm.at[pl.ds(base, EPW)], i0_v)
    pltpu.sync_copy(i1_hbm.at[pl.ds(base, EPW)], i1_v)

    def chunk(j, carry):
        off = j * CHUNK
        cp0 = pltpu.async_copy(g_hbm.at[i0_v.at[pl.ds(off, CHUNK)]], rows0, sem0)
        cp1 = pltpu.async_copy(g_hbm.at[i1_v.at[pl.ds(off, CHUNK)]], rows1, sem1)
        cp0.wait()
        pltpu.sync_copy(rows0, s0_hbm.at[pl.ds(base + off, CHUNK)])
        cp1.wait()
        pltpu.sync_copy(rows1, s1_hbm.at[pl.ds(base + off, CHUNK)])
        return carry

    lax.fori_loop(0, NCHUNK, chunk, 0, unroll=False)


def _sc_gather(G2, idx0, idx1):
    mesh = plsc.VectorSubcoreMesh(
        core_axis_name="c", subcore_axis_name="s",
        num_cores=NC, num_subcores=NS,
    )
    fn = pl.kernel(
        _sc_gather_body,
        out_type=(
            jax.ShapeDtypeStruct((M, 128), jnp.int32),
            jax.ShapeDtypeStruct((M, 128), jnp.int32),
        ),
        mesh=mesh,
        scratch_types=[
            pltpu.VMEM((EPW,), jnp.int32),
            pltpu.VMEM((EPW,), jnp.int32),
            pltpu.VMEM((CHUNK, 128), jnp.int32),
            pltpu.VMEM((CHUNK, 128), jnp.int32),
            pltpu.SemaphoreType.DMA,
            pltpu.SemaphoreType.DMA,
        ],
    )
    return fn(G2, idx0, idx1)


def _edge_body(e_ref, s0_ref, s1_ref, w2_ref, b_ref, lnw_ref, lnb_ref, o_ref):
    e = e_ref[...]
    f = lax.dot_general(
        e, w2_ref[...],
        dimension_numbers=(((1,), (1,)), ((), ())),
        preferred_element_type=jnp.float32,
    )
    u0 = lax.bitcast_convert_type(s0_ref[...], jnp.uint32)
    u1 = lax.bitcast_convert_type(s1_ref[...], jnp.uint32)
    s16 = jnp.uint32(16)
    mask = jnp.uint32(0xFFFF0000)
    lo = (lax.bitcast_convert_type(lax.shift_left(u0, s16), jnp.float32)
          + lax.bitcast_convert_type(lax.shift_left(u1, s16), jnp.float32))
    hi = (lax.bitcast_convert_type(u0 & mask, jnp.float32)
          + lax.bitcast_convert_type(u1 & mask, jnp.float32))
    s = jnp.concatenate([lo, hi], axis=1)
    pre = f + s + b_ref[...]
    act = jnp.where(pre >= 0, pre, 0.01 * pre)
    x = act + e
    mu = jnp.mean(x, axis=1, keepdims=True)
    xc = x - mu
    var = jnp.mean(xc * xc, axis=1, keepdims=True)
    inv = lax.rsqrt(var + 1e-5)
    o_ref[...] = xc * inv * lnw_ref[...] + lnb_ref[...]


def _edge_update(E, S0, S1, W2, b, ln_w, ln_b):
    bm = 1000
    return pl.pallas_call(
        _edge_body,
        grid=(M // bm,),
        in_specs=[
            pl.BlockSpec((bm, D), lambda i: (i, 0)),
            pl.BlockSpec((bm, 128), lambda i: (i, 0)),
            pl.BlockSpec((bm, 128), lambda i: (i, 0)),
            pl.BlockSpec((D, D), lambda i: (0, 0)),
            pl.BlockSpec((1, D), lambda i: (0, 0)),
            pl.BlockSpec((1, D), lambda i: (0, 0)),
            pl.BlockSpec((1, D), lambda i: (0, 0)),
        ],
        out_specs=pl.BlockSpec((bm, D), lambda i: (i, 0)),
        out_shape=jax.ShapeDtypeStruct((M, D), jnp.float32),
    )(E, S0, S1, W2, b, ln_w, ln_b)


@jax.jit
def kernel(H, E, ht, W, b, ln_w, ln_b):
    W1 = W[:, :D]
    W2 = W[:, D:2 * D]
    W3 = W[:, 2 * D:]
    Wc = jnp.concatenate([W1, W3], axis=0)

    G = _node_table(H, Wc)
    G2 = G.reshape(2 * N, 128)

    idx0 = ht[:, 0] * 2
    idx1 = ht[:, 1] * 2 + 1
    S0, S1 = _sc_gather(G2, idx0, idx1)

    b2 = b.reshape(1, D)
    lnw2 = ln_w.reshape(1, D)
    lnb2 = ln_b.reshape(1, D)
    return _edge_update(E, S0, S1, W2, b2, lnw2, lnb2)
